# baseline (speedup 1.0000x reference)
import functools

import jax
import jax.numpy as jnp
import numpy as np
from jax import lax
from jax.experimental import pallas as pl
from jax.experimental.pallas import tpu as pltpu
from jax.experimental.pallas import tpu_sc as plsc

_NUM_KEYPOINTS = 2048
_SAMPLE_RADIUS = 1.6
_NUM_SECTORS = 6
_N_POINTS = 100000
_N_ROIS = 128

_ROWS = 784
_PAD = _ROWS * 128
_BUF_ROWS = (_NUM_SECTORS + 1) * _NUM_KEYPOINTS // 128

_NW = 32
_CHUNK = _PAD // _NW
_NVEC = _CHUNK // 16
_CPAD = 3200
_GSIZE = (_NUM_SECTORS + 1) * _PAD + 128
_GROWS = _GSIZE // 128
_TRASH = (_NUM_SECTORS + 1) * _PAD


def _mask_kernel(px_ref, py_ref, pz_ref, sec_ref,
                 cx_ref, cy_ref, cz_ref, thr_ref, code_ref):
    f32 = jnp.float32
    i32 = jnp.int32
    shape = (_ROWS, 128)
    row_iota = lax.broadcasted_iota(i32, shape, 0)
    col_iota = lax.broadcasted_iota(i32, shape, 1)
    pt_iota = row_iota * 128 + col_iota

    x = px_ref[...]
    y = py_ref[...]
    z = pz_ref[...]
    sec = sec_ref[...]

    def roi_body(r, carry):
        cur_s, tsel = carry
        dx = x - cx_ref[r]
        dy = y - cy_ref[r]
        dz = z - cz_ref[r]
        d2 = (dx * dx + dy * dy) + dz * dz
        s = jnp.sqrt(d2)
        lt = s < cur_s
        cur_s = jnp.where(lt, s, cur_s)
        tsel = jnp.where(lt, thr_ref[r], tsel)
        return cur_s, tsel

    init = (jnp.full(shape, jnp.inf, f32), jnp.zeros(shape, f32))
    cur_s, tsel = lax.fori_loop(0, _N_ROIS, roi_body, init)
    mask = cur_s < tsel

    code = jnp.where(mask, sec, jnp.int32(_NUM_SECTORS + 1))
    anyv = jnp.sum(mask.astype(i32)) > 0
    fb = jnp.where(pt_iota == 0, sec, jnp.int32(_NUM_SECTORS + 1))
    code_ref[...] = jnp.where(anyv, code, fb)


def _sc_count_body(code_hbm, counts_out, code_v, cnt_v):
    i32 = jnp.int32
    wid = lax.axis_index("s") * 2 + lax.axis_index("c")
    base = wid * _CHUNK
    pltpu.sync_copy(code_hbm.at[pl.ds(base, _CHUNK)], code_v)

    zero16 = jnp.zeros((16,), i32)

    def body(j, cvs):
        c = code_v[pl.ds(j * 16, 16)]
        return tuple(
            cv + jnp.where(c == k, jnp.int32(1), jnp.int32(0))
            for k, cv in enumerate(cvs)
        )

    cvs = lax.fori_loop(0, _NVEC, body, (zero16,) * (_NUM_SECTORS + 1))

    iota16 = lax.broadcasted_iota(i32, (16,), 0)
    out_vec = jnp.zeros((16,), i32)
    for k in range(_NUM_SECTORS + 1):
        out_vec = jnp.where(iota16 == k, jnp.sum(cvs[k]), out_vec)
    cnt_v[...] = out_vec
    pltpu.sync_copy(cnt_v.at[pl.ds(0, 8)], counts_out.at[pl.ds(wid * 8, 8)])


def _sc_scatter_body(code_hbm, pxf, pyf, pzf, counts_hbm,
                     xg, yg, zg, ig,
                     code_v, x_v, y_v, z_v, dst2d, iv2d, cnts_v, sem):
    i32 = jnp.int32
    wid = lax.axis_index("s") * 2 + lax.axis_index("c")
    base = wid * _CHUNK

    pltpu.sync_copy(code_hbm.at[pl.ds(base, _CHUNK)], code_v)
    pltpu.sync_copy(pxf.at[pl.ds(base, _CHUNK)], x_v.at[pl.ds(0, _CHUNK)])
    pltpu.sync_copy(pyf.at[pl.ds(base, _CHUNK)], y_v.at[pl.ds(0, _CHUNK)])
    pltpu.sync_copy(pzf.at[pl.ds(base, _CHUNK)], z_v.at[pl.ds(0, _CHUNK)])
    pltpu.sync_copy(counts_hbm, cnts_v)

    iota16 = lax.broadcasted_iota(i32, (16,), 0)
    trash = jnp.int32(_TRASH) + iota16

    offs = []
    for k in range(_NUM_SECTORS + 1):
        a = cnts_v[pl.ds(k * 32, 16)]
        b = cnts_v[pl.ds(k * 32 + 16, 16)]
        off = (jnp.int32(k * _PAD)
               + jnp.sum(jnp.where(iota16 < wid, a, jnp.int32(0)))
               + jnp.sum(jnp.where(iota16 + 16 < wid, b, jnp.int32(0))))
        offs.append(off)

    def fill(j, _):
        dst2d[j // 8, pl.ds((j % 8) * 16, 16)] = trash
        return 0

    lax.fori_loop(0, _CPAD // 16, fill, 0)

    def body(j, offs):
        c = code_v[pl.ds(j * 16, 16)]
        iv = (base + j * 16) + iota16
        dst = trash
        new_offs = []
        for k in range(_NUM_SECTORS + 1):
            m = c == k
            mi = jnp.where(m, jnp.int32(1), jnp.int32(0))
            ranks = plsc.cumsum(mi)
            cnt = jnp.max(ranks)
            dst = jnp.where(m, offs[k] + ranks - 1, dst)
            new_offs.append(offs[k] + cnt)
        dst2d[j // 8, pl.ds((j % 8) * 16, 16)] = dst
        iv2d[j // 8, pl.ds((j % 8) * 16, 16)] = iv
        return tuple(new_offs)

    lax.fori_loop(0, _NVEC, body, tuple(offs))

    copies = []
    for j in range(_CPAD // 128):
        idx = dst2d.at[j]
        copies.append(pltpu.async_copy(x_v.at[pl.ds(j * 128, 128)],
                                       xg.at[idx], sem))
        copies.append(pltpu.async_copy(y_v.at[pl.ds(j * 128, 128)],
                                       yg.at[idx], sem))
        copies.append(pltpu.async_copy(z_v.at[pl.ds(j * 128, 128)],
                                       zg.at[idx], sem))
        copies.append(pltpu.async_copy(iv2d.at[j], ig.at[idx], sem))
    for c in copies:
        c.wait()


def _fps_kernel(xg_ref, yg_ref, zg_ref, ig_ref, counts_ref,
                buf_ref, num_ref, dist_ref):
    f32 = jnp.float32
    i32 = jnp.int32
    lane1 = lax.broadcasted_iota(i32, (1, 128), 1)
    pos8 = (lax.broadcasted_iota(i32, (8, 128), 0) * 128
            + lax.broadcasted_iota(i32, (8, 128), 1))

    cnts = []
    for k in range(_NUM_SECTORS + 1):
        c = counts_ref[0, k]
        for w in range(1, _NW):
            c = c + counts_ref[w, k]
        cnts.append(c)
    total = cnts[0]
    for k in range(1, _NUM_SECTORS + 1):
        total = total + cnts[k]

    nsamps = [jnp.minimum(c, (c * _NUM_KEYPOINTS + total - 1) // total)
              for c in cnts[:_NUM_SECTORS]]
    sector_num = nsamps[0]
    for k in range(1, _NUM_SECTORS):
        sector_num = sector_num + nsamps[k]
    nsamp_fb = jnp.where(sector_num == 0,
                         jnp.minimum(jnp.int32(_NUM_KEYPOINTS), total),
                         jnp.int32(0))
    nsamps.append(nsamp_fb)

    offsets = []
    off = jnp.int32(0)
    for k in range(_NUM_SECTORS + 1):
        offsets.append(off)
        off = off + nsamps[k]
    num_ref[0, 0] = off

    buf_ref[...] = jnp.zeros((_BUF_ROWS, 128), i32)
    big = jnp.int32(_GSIZE)

    def store_at(pos, value):
        prow = pos // 128
        pcol = pos - prow * 128
        cur = buf_ref[pl.ds(prow, 1), :]
        buf_ref[pl.ds(prow, 1), :] = jnp.where(lane1 == pcol, value, cur)

    def fetch_at(ref, grow, gcol):
        r = ref[pl.ds(grow, 1), :]
        return jnp.sum(jnp.where(lane1 == gcol, r, 0))

    def fetch_f_at(ref, grow, gcol):
        r = ref[pl.ds(grow, 1), :]
        return jnp.sum(jnp.where(lane1 == gcol, r, jnp.float32(0.0)))

    for k in range(_NUM_SECTORS + 1):
        base_row = _ROWS * k
        cnt_k = cnts[k]
        ns_k = nsamps[k]
        off_k = offsets[k]
        nblk = (cnt_k + 1023) // 1024

        @pl.when(ns_k > 0)
        def _():
            def initb(b, _):
                rem = cnt_k - b * 1024
                dist_ref[pl.ds(b * 8, 8), :] = jnp.where(
                    pos8 < rem, jnp.float32(1e10), jnp.float32(-1.0))
                return 0

            lax.fori_loop(0, nblk, initb, 0)

            orig0 = fetch_at(ig_ref, base_row, 0)
            store_at(off_k, orig0)
            lx0 = fetch_f_at(xg_ref, base_row, 0)
            ly0 = fetch_f_at(yg_ref, base_row, 0)
            lz0 = fetch_f_at(zg_ref, base_row, 0)

            def body(i, carry):
                lx, ly, lz = carry

                def blk(b, bc):
                    best_m, best_b = bc
                    r0 = base_row + b * 8
                    xb = xg_ref[pl.ds(r0, 8), :]
                    yb = yg_ref[pl.ds(r0, 8), :]
                    zb = zg_ref[pl.ds(r0, 8), :]
                    dx = xb - lx
                    dy = yb - ly
                    dz = zb - lz
                    d = (dx * dx + dy * dy) + dz * dz
                    db = jnp.minimum(dist_ref[pl.ds(b * 8, 8), :], d)
                    dist_ref[pl.ds(b * 8, 8), :] = db
                    mb = jnp.max(db)
                    upd = mb > best_m
                    best_m = jnp.where(upd, mb, best_m)
                    best_b = jnp.where(upd, b, best_b)
                    return best_m, best_b

                best_m, best_b = lax.fori_loop(
                    0, nblk, blk, (jnp.float32(-2.0), jnp.int32(0)))

                db = dist_ref[pl.ds(best_b * 8, 8), :]
                inb = jnp.min(jnp.where(db == best_m, pos8, jnp.int32(8192)))
                cp = best_b * 1024 + inb
                grow = base_row + cp // 128
                gcol = cp - (cp // 128) * 128
                orig = fetch_at(ig_ref, grow, gcol)
                store_at(off_k + i, orig)
                nlx = fetch_f_at(xg_ref, grow, gcol)
                nly = fetch_f_at(yg_ref, grow, gcol)
                nlz = fetch_f_at(zg_ref, grow, gcol)
                return nlx, nly, nlz

            lax.fori_loop(1, ns_k, body, (lx0, ly0, lz0))


@jax.jit
def kernel(points, rois):
    f32 = jnp.float32
    i32 = jnp.int32

    sector_size = np.pi * 2.0 / _NUM_SECTORS
    angles = jnp.arctan2(points[:, 1], points[:, 0]) + np.pi
    sector = jnp.clip(jnp.floor(angles / sector_size), 0, _NUM_SECTORS)
    sector = sector.astype(i32)

    cz_shift = rois[:, 2] + rois[:, 5] / 2.0
    half = rois[:, 3:6] / 2.0
    thr = jnp.sqrt((half[:, 0] ** 2 + half[:, 1] ** 2) + half[:, 2] ** 2) \
        + jnp.float32(_SAMPLE_RADIUS)

    pad = _PAD - _N_POINTS
    pxf = jnp.pad(points[:, 0], (0, pad), constant_values=1e9)
    pyf = jnp.pad(points[:, 1], (0, pad), constant_values=1e9)
    pzf = jnp.pad(points[:, 2], (0, pad), constant_values=1e9)
    secf = jnp.pad(sector, (0, pad), constant_values=_NUM_SECTORS + 1)

    smem = pl.BlockSpec(memory_space=pltpu.SMEM)

    code2d = pl.pallas_call(
        _mask_kernel,
        in_specs=[pl.BlockSpec((_ROWS, 128), lambda: (0, 0))] * 4 +
                 [smem] * 4,
        out_specs=pl.BlockSpec((_ROWS, 128), lambda: (0, 0)),
        out_shape=jax.ShapeDtypeStruct((_ROWS, 128), i32),
    )(pxf.reshape(_ROWS, 128), pyf.reshape(_ROWS, 128),
      pzf.reshape(_ROWS, 128), secf.reshape(_ROWS, 128),
      rois[:, 0].astype(f32), rois[:, 1].astype(f32), cz_shift, thr)

    codef = code2d.reshape(-1)

    mesh = plsc.VectorSubcoreMesh(core_axis_name="c", subcore_axis_name="s")

    counts = pl.kernel(
        _sc_count_body,
        out_type=jax.ShapeDtypeStruct((_NW * 8,), i32),
        mesh=mesh,
        scratch_types=[pltpu.VMEM((_CHUNK,), i32),
                       pltpu.VMEM((16,), i32)],
        compiler_params=pltpu.CompilerParams(needs_layout_passes=False),
    )(codef)
    counts = counts.reshape(_NW, 8)

    xg, yg, zg, ig = pl.kernel(
        _sc_scatter_body,
        out_type=[jax.ShapeDtypeStruct((_GSIZE,), f32),
                  jax.ShapeDtypeStruct((_GSIZE,), f32),
                  jax.ShapeDtypeStruct((_GSIZE,), f32),
                  jax.ShapeDtypeStruct((_GSIZE,), i32)],
        mesh=mesh,
        scratch_types=[pltpu.VMEM((_CHUNK,), i32),
                       pltpu.VMEM((_CPAD,), f32),
                       pltpu.VMEM((_CPAD,), f32),
                       pltpu.VMEM((_CPAD,), f32),
                       pltpu.VMEM((_CPAD // 128, 128), i32),
                       pltpu.VMEM((_CPAD // 128, 128), i32),
                       pltpu.VMEM((_NW * 8,), i32),
                       pltpu.SemaphoreType.DMA],
        compiler_params=pltpu.CompilerParams(needs_layout_passes=False),
    )(codef, pxf, pyf, pzf, counts.T.reshape(-1))

    buf, num = pl.pallas_call(
        _fps_kernel,
        in_specs=[pl.BlockSpec((_GROWS, 128), lambda: (0, 0))] * 4 + [smem],
        out_specs=[pl.BlockSpec((_BUF_ROWS, 128), lambda: (0, 0)), smem],
        out_shape=[jax.ShapeDtypeStruct((_BUF_ROWS, 128), i32),
                   jax.ShapeDtypeStruct((1, 1), i32)],
        scratch_shapes=[pltpu.VMEM((_ROWS, 128), f32)],
    )(xg.reshape(_GROWS, 128), yg.reshape(_GROWS, 128),
      zg.reshape(_GROWS, 128), ig.reshape(_GROWS, 128), counts)

    n = num[0, 0]
    idx = buf.reshape(-1)[jnp.arange(_NUM_KEYPOINTS, dtype=i32) % n]
    return jnp.take(points, idx, axis=0)


# speedup vs baseline: 12.7285x; 12.7285x over previous
import functools

import jax
import jax.numpy as jnp
import numpy as np
from jax import lax
from jax.experimental import pallas as pl
from jax.experimental.pallas import tpu as pltpu
from jax.experimental.pallas import tpu_sc as plsc

_NUM_KEYPOINTS = 2048
_SAMPLE_RADIUS = 1.6
_NUM_SECTORS = 6
_N_POINTS = 100000
_N_ROIS = 128

_ROWS = 784
_PAD = _ROWS * 128
_BUF_ROWS = (_NUM_SECTORS + 1) * _NUM_KEYPOINTS // 128

_NW = 32
_CHUNK = _PAD // _NW
_NVEC = _CHUNK // 16
_CPAD = 3216
_SPAD = _PAD + 256
_SROWS = _SPAD // 128
_GSIZE = (_NUM_SECTORS + 1) * _SPAD + 768
_GROWS = _GSIZE // 128


def _mask_kernel(px_ref, py_ref, pz_ref, sec_ref,
                 cx_ref, cy_ref, cz_ref, thr_ref, code_ref):
    f32 = jnp.float32
    i32 = jnp.int32
    shape = (_ROWS, 128)
    row_iota = lax.broadcasted_iota(i32, shape, 0)
    col_iota = lax.broadcasted_iota(i32, shape, 1)
    pt_iota = row_iota * 128 + col_iota

    x = px_ref[...]
    y = py_ref[...]
    z = pz_ref[...]
    sec = sec_ref[...]

    def roi_body(r, carry):
        cur_s, tsel = carry
        dx = x - cx_ref[r]
        dy = y - cy_ref[r]
        dz = z - cz_ref[r]
        d2 = (dx * dx + dy * dy) + dz * dz
        s = jnp.sqrt(d2)
        lt = s < cur_s
        cur_s = jnp.where(lt, s, cur_s)
        tsel = jnp.where(lt, thr_ref[r], tsel)
        return cur_s, tsel

    init = (jnp.full(shape, jnp.inf, f32), jnp.zeros(shape, f32))
    cur_s, tsel = lax.fori_loop(0, _N_ROIS, roi_body, init)
    mask = cur_s < tsel

    code = jnp.where(mask, sec, jnp.int32(_NUM_SECTORS + 1))
    anyv = jnp.sum(mask.astype(i32)) > 0
    fb = jnp.where(pt_iota == 0, sec, jnp.int32(_NUM_SECTORS + 1))
    code_ref[...] = jnp.where(anyv, code, fb)


def _sc_count_body(code_hbm, counts_out, code_v, cnt_v):
    i32 = jnp.int32
    wid = lax.axis_index("s") * 2 + lax.axis_index("c")
    base = wid * _CHUNK
    pltpu.sync_copy(code_hbm.at[pl.ds(pl.multiple_of(base, 8), _CHUNK)], code_v)

    zero16 = jnp.zeros((16,), i32)

    def body(j, cvs):
        c = code_v[pl.ds(j * 16, 16)]
        return tuple(
            cv + jnp.where(c == k, jnp.int32(1), jnp.int32(0))
            for k, cv in enumerate(cvs)
        )

    cvs = lax.fori_loop(0, _NVEC, body, (zero16,) * (_NUM_SECTORS + 1))

    iota16 = lax.broadcasted_iota(i32, (16,), 0)
    out_vec = jnp.zeros((16,), i32)
    for k in range(_NUM_SECTORS + 1):
        out_vec = jnp.where(iota16 == k, jnp.sum(cvs[k]), out_vec)
    cnt_v[...] = out_vec
    pltpu.sync_copy(cnt_v.at[pl.ds(0, 8)], counts_out.at[pl.ds(pl.multiple_of(wid * 8, 8), 8)])


def _sc_scatter_body(code_hbm, pxf, pyf, pzf, counts_hbm,
                     xg, yg, zg, ig,
                     code_v, x_v, y_v, z_v,
                     cx_v, cy_v, cz_v, ci_v, cnts_v, sem):
    i32 = jnp.int32
    f32 = jnp.float32
    wid = lax.axis_index("s") * 2 + lax.axis_index("c")
    base = wid * _CHUNK

    pltpu.sync_copy(code_hbm.at[pl.ds(pl.multiple_of(base, 8), _CHUNK)], code_v)
    pltpu.sync_copy(pxf.at[pl.ds(pl.multiple_of(base, 8), _CHUNK)], x_v)
    pltpu.sync_copy(pyf.at[pl.ds(pl.multiple_of(base, 8), _CHUNK)], y_v)
    pltpu.sync_copy(pzf.at[pl.ds(pl.multiple_of(base, 8), _CHUNK)], z_v)
    pltpu.sync_copy(counts_hbm, cnts_v)

    iota16 = lax.broadcasted_iota(i32, (16,), 0)

    def r8(v):
        return (v + 7) & ~7

    goffs, pcnts = [], []
    loffs = [jnp.int32(0)]
    for k in range(_NUM_SECTORS + 1):
        a = r8(cnts_v[pl.ds(k * 32, 16)])
        b = r8(cnts_v[pl.ds(k * 32 + 16, 16)])
        goffs.append(jnp.int32(k * _SPAD)
                     + jnp.sum(jnp.where(iota16 < wid, a, jnp.int32(0)))
                     + jnp.sum(jnp.where(iota16 + 16 < wid, b, jnp.int32(0))))
        own = (jnp.sum(jnp.where(iota16 == wid, a, jnp.int32(0)))
               + jnp.sum(jnp.where(iota16 + 16 == wid, b, jnp.int32(0))))
        pcnts.append(own)
        loffs.append(loffs[-1] + own)

    zf = jnp.zeros((16,), f32)
    neg1 = jnp.full((16,), -1, i32)

    def fill(j, _):
        cx_v[pl.ds(j * 16, 16)] = zf
        cy_v[pl.ds(j * 16, 16)] = zf
        cz_v[pl.ds(j * 16, 16)] = zf
        ci_v[pl.ds(j * 16, 16)] = neg1
        return 0

    lax.fori_loop(0, _CPAD // 16, fill, 0)

    def body(j, offs):
        c = code_v[pl.ds(j * 16, 16)]
        xv = x_v[pl.ds(j * 16, 16)]
        yv = y_v[pl.ds(j * 16, 16)]
        zv = z_v[pl.ds(j * 16, 16)]
        iv = (base + j * 16) + iota16
        new_offs = []
        for k in range(_NUM_SECTORS + 1):
            m = c == k
            mi = jnp.where(m, jnp.int32(1), jnp.int32(0))
            ranks = plsc.cumsum(mi)
            pos = offs[k] + ranks - 1
            plsc.store_scatter(cx_v, [pos], xv, mask=m)
            plsc.store_scatter(cy_v, [pos], yv, mask=m)
            plsc.store_scatter(cz_v, [pos], zv, mask=m)
            plsc.store_scatter(ci_v, [pos], iv, mask=m)
            new_offs.append(offs[k] + jnp.max(ranks))
        return tuple(new_offs)

    lax.fori_loop(0, _NVEC, body, tuple(loffs[:-1]))

    def ship(local, ghbm):
        for k in range(_NUM_SECTORS + 1):
            lo, go, pc = loffs[k], goffs[k], pcnts[k]

            def big(j, _):
                pltpu.async_copy(local.at[pl.ds(pl.multiple_of(lo + j * 64, 8), 64)],
                                 ghbm.at[pl.ds(pl.multiple_of(go + j * 64, 8), 64)], sem)
                return 0

            def small(j, _):
                pltpu.async_copy(local.at[pl.ds(pl.multiple_of(lo + j * 8, 8), 8)],
                                 ghbm.at[pl.ds(pl.multiple_of(go + j * 8, 8), 8)], sem)
                return 0

            n64 = pc // 64
            lax.fori_loop(0, n64, big, 0)
            lax.fori_loop(n64 * 8, pc // 8, small, 0)

    def drain(local, ghbm):
        for k in range(_NUM_SECTORS + 1):
            lo, go, pc = loffs[k], goffs[k], pcnts[k]

            def big(j, _):
                pltpu.make_async_copy(local.at[pl.ds(pl.multiple_of(lo + j * 64, 8), 64)],
                                      ghbm.at[pl.ds(pl.multiple_of(go + j * 64, 8), 64)],
                                      sem).wait()
                return 0

            def small(j, _):
                pltpu.make_async_copy(local.at[pl.ds(pl.multiple_of(lo + j * 8, 8), 8)],
                                      ghbm.at[pl.ds(pl.multiple_of(go + j * 8, 8), 8)],
                                      sem).wait()
                return 0

            n64 = pc // 64
            lax.fori_loop(0, n64, big, 0)
            lax.fori_loop(n64 * 8, pc // 8, small, 0)

    for local, ghbm in ((cx_v, xg), (cy_v, yg), (cz_v, zg), (ci_v, ig)):
        ship(local, ghbm)
    for local, ghbm in ((cx_v, xg), (cy_v, yg), (cz_v, zg), (ci_v, ig)):
        drain(local, ghbm)


def _fps_kernel(xg_ref, yg_ref, zg_ref, ig_ref, counts_ref,
                buf_ref, num_ref, dist_ref):
    f32 = jnp.float32
    i32 = jnp.int32
    lane1 = lax.broadcasted_iota(i32, (1, 128), 1)
    pos8 = (lax.broadcasted_iota(i32, (8, 128), 0) * 128
            + lax.broadcasted_iota(i32, (8, 128), 1))

    cnts = []
    pcnts = []
    for k in range(_NUM_SECTORS + 1):
        c = counts_ref[0, k]
        p = (counts_ref[0, k] + 7) // 8 * 8
        for w in range(1, _NW):
            c = c + counts_ref[w, k]
            p = p + (counts_ref[w, k] + 7) // 8 * 8
        cnts.append(c)
        pcnts.append(p)
    total = cnts[0]
    for k in range(1, _NUM_SECTORS + 1):
        total = total + cnts[k]

    nsamps = [jnp.minimum(c, (c * _NUM_KEYPOINTS + total - 1) // total)
              for c in cnts[:_NUM_SECTORS]]
    sector_num = nsamps[0]
    for k in range(1, _NUM_SECTORS):
        sector_num = sector_num + nsamps[k]
    nsamp_fb = jnp.where(sector_num == 0,
                         jnp.minimum(jnp.int32(_NUM_KEYPOINTS), total),
                         jnp.int32(0))
    nsamps.append(nsamp_fb)

    offsets = []
    off = jnp.int32(0)
    for k in range(_NUM_SECTORS + 1):
        offsets.append(off)
        off = off + nsamps[k]
    num_ref[0, 0] = off

    buf_ref[...] = jnp.zeros((_BUF_ROWS, 128), i32)
    big = jnp.int32(_GSIZE)

    def store_at(pos, value):
        prow = pos // 128
        pcol = pos - prow * 128
        cur = buf_ref[pl.ds(prow, 1), :]
        buf_ref[pl.ds(prow, 1), :] = jnp.where(lane1 == pcol, value, cur)

    def fetch_at(ref, grow, gcol):
        r = ref[pl.ds(grow, 1), :]
        return jnp.sum(jnp.where(lane1 == gcol, r, 0))

    def fetch_f_at(ref, grow, gcol):
        r = ref[pl.ds(grow, 1), :]
        return jnp.sum(jnp.where(lane1 == gcol, r, jnp.float32(0.0)))

    for k in range(_NUM_SECTORS + 1):
        base_row = _SROWS * k
        pcnt_k = pcnts[k]
        ns_k = nsamps[k]
        off_k = offsets[k]
        nblk = (pcnt_k + 1023) // 1024

        @pl.when(ns_k > 0)
        def _():
            def initb(b, _):
                rem = pcnt_k - b * 1024
                igb = ig_ref[pl.ds(base_row + b * 8, 8), :]
                live = (pos8 < rem) & (igb >= 0)
                dist_ref[pl.ds(b * 8, 8), :] = jnp.where(
                    live, jnp.float32(1e10), jnp.float32(-1.0))
                return 0

            lax.fori_loop(0, nblk, initb, 0)

            orig0 = fetch_at(ig_ref, base_row, 0)
            store_at(off_k, orig0)
            lx0 = fetch_f_at(xg_ref, base_row, 0)
            ly0 = fetch_f_at(yg_ref, base_row, 0)
            lz0 = fetch_f_at(zg_ref, base_row, 0)

            def body(i, carry):
                lx, ly, lz = carry

                def blk(b, bc):
                    best_m, best_b = bc
                    r0 = base_row + b * 8
                    xb = xg_ref[pl.ds(r0, 8), :]
                    yb = yg_ref[pl.ds(r0, 8), :]
                    zb = zg_ref[pl.ds(r0, 8), :]
                    dx = xb - lx
                    dy = yb - ly
                    dz = zb - lz
                    d = (dx * dx + dy * dy) + dz * dz
                    db0 = dist_ref[pl.ds(b * 8, 8), :]
                    db = jnp.where(db0 < 0.0, db0, jnp.minimum(db0, d))
                    dist_ref[pl.ds(b * 8, 8), :] = db
                    mb = jnp.max(db)
                    upd = mb > best_m
                    best_m = jnp.where(upd, mb, best_m)
                    best_b = jnp.where(upd, b, best_b)
                    return best_m, best_b

                best_m, best_b = lax.fori_loop(
                    0, nblk, blk, (jnp.float32(-2.0), jnp.int32(0)))

                db = dist_ref[pl.ds(best_b * 8, 8), :]
                inb = jnp.min(jnp.where(db == best_m, pos8, jnp.int32(8192)))
                cp = best_b * 1024 + inb
                grow = base_row + cp // 128
                gcol = cp - (cp // 128) * 128
                orig = fetch_at(ig_ref, grow, gcol)
                store_at(off_k + i, orig)
                nlx = fetch_f_at(xg_ref, grow, gcol)
                nly = fetch_f_at(yg_ref, grow, gcol)
                nlz = fetch_f_at(zg_ref, grow, gcol)
                return nlx, nly, nlz

            lax.fori_loop(1, ns_k, body, (lx0, ly0, lz0))


@jax.jit
def kernel(points, rois):
    f32 = jnp.float32
    i32 = jnp.int32

    sector_size = np.pi * 2.0 / _NUM_SECTORS
    angles = jnp.arctan2(points[:, 1], points[:, 0]) + np.pi
    sector = jnp.clip(jnp.floor(angles / sector_size), 0, _NUM_SECTORS)
    sector = sector.astype(i32)

    cz_shift = rois[:, 2] + rois[:, 5] / 2.0
    half = rois[:, 3:6] / 2.0
    thr = jnp.sqrt((half[:, 0] ** 2 + half[:, 1] ** 2) + half[:, 2] ** 2) \
        + jnp.float32(_SAMPLE_RADIUS)

    pad = _PAD - _N_POINTS
    pxf = jnp.pad(points[:, 0], (0, pad), constant_values=1e9)
    pyf = jnp.pad(points[:, 1], (0, pad), constant_values=1e9)
    pzf = jnp.pad(points[:, 2], (0, pad), constant_values=1e9)
    secf = jnp.pad(sector, (0, pad), constant_values=_NUM_SECTORS + 1)

    smem = pl.BlockSpec(memory_space=pltpu.SMEM)

    code2d = pl.pallas_call(
        _mask_kernel,
        in_specs=[pl.BlockSpec((_ROWS, 128), lambda: (0, 0))] * 4 +
                 [smem] * 4,
        out_specs=pl.BlockSpec((_ROWS, 128), lambda: (0, 0)),
        out_shape=jax.ShapeDtypeStruct((_ROWS, 128), i32),
    )(pxf.reshape(_ROWS, 128), pyf.reshape(_ROWS, 128),
      pzf.reshape(_ROWS, 128), secf.reshape(_ROWS, 128),
      rois[:, 0].astype(f32), rois[:, 1].astype(f32), cz_shift, thr)

    codef = code2d.reshape(-1)

    mesh = plsc.VectorSubcoreMesh(core_axis_name="c", subcore_axis_name="s")

    counts = pl.kernel(
        _sc_count_body,
        out_type=jax.ShapeDtypeStruct((_NW * 8,), i32),
        mesh=mesh,
        scratch_types=[pltpu.VMEM((_CHUNK,), i32),
                       pltpu.VMEM((16,), i32)],
        compiler_params=pltpu.CompilerParams(needs_layout_passes=False),
    )(codef)
    counts = counts.reshape(_NW, 8)

    xg, yg, zg, ig = pl.kernel(
        _sc_scatter_body,
        out_type=[jax.ShapeDtypeStruct((_GSIZE,), f32),
                  jax.ShapeDtypeStruct((_GSIZE,), f32),
                  jax.ShapeDtypeStruct((_GSIZE,), f32),
                  jax.ShapeDtypeStruct((_GSIZE,), i32)],
        mesh=mesh,
        scratch_types=[pltpu.VMEM((_CHUNK,), i32),
                       pltpu.VMEM((_CHUNK,), f32),
                       pltpu.VMEM((_CHUNK,), f32),
                       pltpu.VMEM((_CHUNK,), f32),
                       pltpu.VMEM((_CPAD,), f32),
                       pltpu.VMEM((_CPAD,), f32),
                       pltpu.VMEM((_CPAD,), f32),
                       pltpu.VMEM((_CPAD,), i32),
                       pltpu.VMEM((_NW * 8,), i32),
                       pltpu.SemaphoreType.DMA],
        compiler_params=pltpu.CompilerParams(needs_layout_passes=False),
    )(codef, pxf, pyf, pzf, counts.T.reshape(-1))

    buf, num = pl.pallas_call(
        _fps_kernel,
        in_specs=[pl.BlockSpec((_GROWS, 128), lambda: (0, 0))] * 4 + [smem],
        out_specs=[pl.BlockSpec((_BUF_ROWS, 128), lambda: (0, 0)), smem],
        out_shape=[jax.ShapeDtypeStruct((_BUF_ROWS, 128), i32),
                   jax.ShapeDtypeStruct((1, 1), i32)],
        scratch_shapes=[pltpu.VMEM((792, 128), f32)],
    )(xg.reshape(_GROWS, 128), yg.reshape(_GROWS, 128),
      zg.reshape(_GROWS, 128), ig.reshape(_GROWS, 128), counts)

    n = num[0, 0]
    idx = buf.reshape(-1)[jnp.arange(_NUM_KEYPOINTS, dtype=i32) % n]
    return jnp.take(points, idx, axis=0)
